# Initial kernel scaffold; baseline (speedup 1.0000x reference)
import functools

import jax
import jax.numpy as jnp
from jax import lax
from jax.experimental import pallas as pl
from jax.experimental.pallas import tpu as pltpu
from jax.experimental.pallas import tpu_sc as plsc

NC = 2
NS = 16
NW = NC * NS
CB = 128
GSEG = 64


def _sc_degree(dstp, acc_n, cpw):
    rows_per_tile = acc_n // NS
    mesh = plsc.VectorSubcoreMesh(core_axis_name="c", subcore_axis_name="s")

    def body(dst_hbm, ones_hbm, zero_hbm, out_hbm, dst_v, ones_v, acc_sh, sem):
        c = lax.axis_index("c")
        s = lax.axis_index("s")
        w = s * NC + c
        r0 = s * rows_per_tile
        pltpu.sync_copy(zero_hbm.at[pl.ds(r0, rows_per_tile)],
                        acc_sh.at[pl.ds(r0, rows_per_tile)])
        pltpu.sync_copy(ones_hbm, ones_v)
        plsc.subcore_barrier()
        base0 = w * cpw * CB

        def step(i, carry):
            pltpu.sync_copy(dst_hbm.at[pl.ds(base0 + i * CB, CB)], dst_v)
            pltpu.sync_copy(ones_v, acc_sh.at[dst_v], add=True)
            return carry

        lax.fori_loop(0, cpw, step, 0)
        plsc.subcore_barrier()
        pltpu.sync_copy(acc_sh.at[pl.ds(r0, rows_per_tile)],
                        out_hbm.at[c, pl.ds(r0, rows_per_tile)])

    ones = jnp.ones((CB, 16), jnp.float32)
    zero = jnp.zeros((acc_n, 16), jnp.float32)
    call = pl.kernel(
        body,
        out_type=jax.ShapeDtypeStruct((NC, acc_n, 16), jnp.float32),
        mesh=mesh,
        scratch_types=[
            pltpu.VMEM((CB,), jnp.int32),
            pltpu.VMEM((CB, 16), jnp.float32),
            pltpu.VMEM_SHARED((acc_n, 16), jnp.float32),
            pltpu.SemaphoreType.DMA,
        ],
    )
    return call(dstp, ones, zero)


def _sc_scatter(ms, srcp, dstp, zero, acc_n, cpw, h):
    rows_per_tile = acc_n // NS
    mesh = plsc.VectorSubcoreMesh(core_axis_name="c", subcore_axis_name="s")

    def body(ms_hbm, src_hbm, dst_hbm, zero_hbm, out_hbm,
             src_v, dst_v, rows_v, acc_sh, sem):
        c = lax.axis_index("c")
        s = lax.axis_index("s")
        w = s * NC + c
        r0 = s * rows_per_tile
        pltpu.sync_copy(zero_hbm.at[pl.ds(r0, rows_per_tile)],
                        acc_sh.at[pl.ds(r0, rows_per_tile)])
        plsc.subcore_barrier()
        base0 = w * cpw * CB

        def step(i, carry):
            base = base0 + i * CB
            pltpu.sync_copy(src_hbm.at[pl.ds(base, CB)], src_v)
            pltpu.sync_copy(dst_hbm.at[pl.ds(base, CB)], dst_v)
            pltpu.async_copy(ms_hbm.at[src_v], rows_v, sem).wait()
            pltpu.sync_copy(rows_v, acc_sh.at[dst_v], add=True)
            return carry

        lax.fori_loop(0, cpw, step, 0)
        plsc.subcore_barrier()
        pltpu.sync_copy(acc_sh.at[pl.ds(r0, rows_per_tile)],
                        out_hbm.at[c, pl.ds(r0, rows_per_tile)])

    call = pl.kernel(
        body,
        out_type=jax.ShapeDtypeStruct((NC, acc_n, h), jnp.float32),
        mesh=mesh,
        scratch_types=[
            pltpu.VMEM((CB,), jnp.int32),
            pltpu.VMEM((CB,), jnp.int32),
            pltpu.VMEM((CB, h), jnp.float32),
            pltpu.VMEM_SHARED((acc_n, h), jnp.float32),
            pltpu.SemaphoreType.DMA,
        ],
    )
    return call(ms, srcp, dstp, zero)


def _tc_init(degp, x, w0, n):
    def body(degp_ref, x_ref, w0_ref, dinv_ref, ms_ref):
        deg = degp_ref[0] + degp_ref[1]
        dinv = lax.rsqrt(deg[:n, 0:1] + 1.0)
        dinv_ref[...] = dinv
        ms_ref[...] = jnp.dot(x_ref[...] * dinv, w0_ref[...],
                              preferred_element_type=jnp.float32)

    h = x.shape[1]
    return pl.pallas_call(
        body,
        out_shape=(jax.ShapeDtypeStruct((n, 1), jnp.float32),
                   jax.ShapeDtypeStruct((n, h), jnp.float32)),
    )(degp, x, w0)


def _layer_post(p_ref, ms_ref, h_ref, dinv_ref, bc_ref, g_ref, b_ref, n):
    ms = ms_ref[...]
    dinv = dinv_ref[...]
    agg = (p_ref[0, :n] + p_ref[1, :n] + ms) * dinv + bc_ref[...]
    a = jnp.maximum(agg, 0.0)
    mu = jnp.mean(a, axis=0, keepdims=True)
    var = jnp.mean((a - mu) ** 2, axis=0, keepdims=True)
    an = (a - mu) * (g_ref[...] * lax.rsqrt(var + 1e-5)) + b_ref[...]
    return an + h_ref[...]


def _tc_layer(p, ms, hprev, dinv, bci, gi, bi, wnext, n):
    def body(p_ref, ms_ref, h_ref, dinv_ref, bc_ref, g_ref, b_ref, wn_ref,
             hout_ref, msout_ref):
        hn = _layer_post(p_ref, ms_ref, h_ref, dinv_ref, bc_ref, g_ref, b_ref, n)
        hout_ref[...] = hn
        msout_ref[...] = jnp.dot(hn * dinv_ref[...], wn_ref[...],
                                 preferred_element_type=jnp.float32)

    h = ms.shape[1]
    return pl.pallas_call(
        body,
        out_shape=(jax.ShapeDtypeStruct((n, h), jnp.float32),
                   jax.ShapeDtypeStruct((n, h), jnp.float32)),
    )(p, ms, hprev, dinv, bci, gi, bi, wnext)


def _tc_final(p, ms, hprev, dinv, bci, gi, bi, batch_row, w1p, b1p, w2p, b2p, n):
    def body(p_ref, ms_ref, h_ref, dinv_ref, bc_ref, g_ref, b_ref,
             batch_ref, w1_ref, b1_ref, w2_ref, b2_ref, out_ref):
        hn = _layer_post(p_ref, ms_ref, h_ref, dinv_ref, bc_ref, g_ref, b_ref, n)
        seg = lax.broadcasted_iota(jnp.int32, (GSEG, n), 0)
        m = (batch_ref[...] == seg).astype(jnp.float32)
        sums = jnp.dot(m, hn, preferred_element_type=jnp.float32)
        cnt = jnp.sum(m, axis=1, keepdims=True)
        pooled = sums / jnp.maximum(cnt, 1.0)
        z = jnp.maximum(
            jnp.dot(pooled, w1_ref[...], preferred_element_type=jnp.float32)
            + b1_ref[...], 0.0)
        out_ref[...] = jnp.dot(z, w2_ref[...],
                               preferred_element_type=jnp.float32) + b2_ref[...]

    return pl.pallas_call(
        body,
        out_shape=jax.ShapeDtypeStruct((GSEG, 128), jnp.float32),
    )(p, ms, hprev, dinv, bci, gi, bi, batch_row, w1p, b1p, w2p, b2p)


def kernel(x, edge_index, batch, Wc, bc, gamma, beta, W1, b1, W2, b2):
    n, h = x.shape
    e = edge_index.shape[1]
    nlayers = Wc.shape[0]
    hh = W1.shape[1]
    nout = W2.shape[1]

    acc_n = n + 16
    cpw = -(-e // (NW * CB))
    ep = NW * CB * cpw
    pad = ep - e
    srcp = jnp.concatenate([edge_index[0], jnp.zeros((pad,), jnp.int32)])
    dstp = jnp.concatenate([edge_index[1], jnp.full((pad,), n, jnp.int32)])
    zero_acc = jnp.zeros((acc_n, h), jnp.float32)

    w1p = jnp.zeros((h, 128), jnp.float32).at[:, :hh].set(W1)
    b1p = jnp.zeros((1, 128), jnp.float32).at[0, :hh].set(b1)
    w2p = jnp.zeros((128, 128), jnp.float32).at[:hh, :nout].set(W2)
    b2p = jnp.zeros((1, 128), jnp.float32).at[0, :nout].set(b2)
    batch_row = batch.reshape(1, n)

    degp = _sc_degree(dstp, acc_n, cpw)
    dinv, ms = _tc_init(degp, x, Wc[0], n)

    hcur = x
    for i in range(nlayers - 1):
        p = _sc_scatter(ms, srcp, dstp, zero_acc, acc_n, cpw, h)
        hcur, ms = _tc_layer(p, ms, hcur, dinv, bc[i].reshape(1, h),
                             gamma[i].reshape(1, h), beta[i].reshape(1, h),
                             Wc[i + 1], n)

    p = _sc_scatter(ms, srcp, dstp, zero_acc, acc_n, cpw, h)
    out_full = _tc_final(p, ms, hcur, dinv, bc[-1].reshape(1, h),
                         gamma[-1].reshape(1, h), beta[-1].reshape(1, h),
                         batch_row, w1p, b1p, w2p, b2p, n)
    return out_full[:, :nout]


# speedup vs baseline: 6.7502x; 6.7502x over previous
import functools

import jax
import jax.numpy as jnp
from jax import lax
from jax.experimental import pallas as pl
from jax.experimental.pallas import tpu as pltpu
from jax.experimental.pallas import tpu_sc as plsc

NC = 2
NS = 16
NW = NC * NS
CB = 128
GSEG = 64


def _sc_degree(dstp, acc_n, cpw, h):
    rows_per_tile = acc_n // NS
    mesh = plsc.VectorSubcoreMesh(core_axis_name="c", subcore_axis_name="s")

    def body(dst_hbm, ones_hbm, zero_hbm, out_hbm, dst_v, ones_v, acc_sh, sem):
        c = lax.axis_index("c")
        s = lax.axis_index("s")
        w = s * NC + c
        r0 = s * rows_per_tile
        pltpu.sync_copy(zero_hbm.at[pl.ds(r0, rows_per_tile)],
                        acc_sh.at[pl.ds(r0, rows_per_tile)])
        pltpu.sync_copy(ones_hbm, ones_v)
        plsc.subcore_barrier()
        base0 = w * cpw * CB

        def step(i, carry):
            pltpu.sync_copy(dst_hbm.at[pl.ds(base0 + i * CB, CB)], dst_v)
            pltpu.sync_copy(ones_v, acc_sh.at[dst_v], add=True)
            return carry

        lax.fori_loop(0, cpw, step, 0)
        plsc.subcore_barrier()
        pltpu.sync_copy(acc_sh.at[pl.ds(r0, rows_per_tile)],
                        out_hbm.at[c, pl.ds(r0, rows_per_tile)])

    ones = jnp.ones((CB, h), jnp.float32)
    zero = jnp.zeros((acc_n, h), jnp.float32)
    call = pl.kernel(
        body,
        out_type=jax.ShapeDtypeStruct((NC, acc_n, h), jnp.float32),
        mesh=mesh,
        scratch_types=[
            pltpu.VMEM((CB,), jnp.int32),
            pltpu.VMEM((CB, h), jnp.float32),
            pltpu.VMEM_SHARED((acc_n, h), jnp.float32),
            pltpu.SemaphoreType.DMA,
        ],
    )
    return call(dstp, ones, zero)


def _sc_scatter(ms, srcp, dstp, zero, acc_n, cpw, h):
    rows_per_tile = acc_n // NS
    mesh = plsc.VectorSubcoreMesh(core_axis_name="c", subcore_axis_name="s")

    def body(ms_hbm, src_hbm, dst_hbm, zero_hbm, out_hbm,
             src_v, dst_v, rows_v, acc_sh, sem):
        c = lax.axis_index("c")
        s = lax.axis_index("s")
        w = s * NC + c
        r0 = s * rows_per_tile
        pltpu.sync_copy(zero_hbm.at[pl.ds(r0, rows_per_tile)],
                        acc_sh.at[pl.ds(r0, rows_per_tile)])
        plsc.subcore_barrier()
        base0 = w * cpw * CB

        def step(i, carry):
            base = base0 + i * CB
            pltpu.sync_copy(src_hbm.at[pl.ds(base, CB)], src_v)
            pltpu.sync_copy(dst_hbm.at[pl.ds(base, CB)], dst_v)
            pltpu.async_copy(ms_hbm.at[src_v], rows_v, sem).wait()
            pltpu.sync_copy(rows_v, acc_sh.at[dst_v], add=True)
            return carry

        lax.fori_loop(0, cpw, step, 0)
        plsc.subcore_barrier()
        pltpu.sync_copy(acc_sh.at[pl.ds(r0, rows_per_tile)],
                        out_hbm.at[c, pl.ds(r0, rows_per_tile)])

    call = pl.kernel(
        body,
        out_type=jax.ShapeDtypeStruct((NC, acc_n, h), jnp.float32),
        mesh=mesh,
        scratch_types=[
            pltpu.VMEM((CB,), jnp.int32),
            pltpu.VMEM((CB,), jnp.int32),
            pltpu.VMEM((CB, h), jnp.float32),
            pltpu.VMEM_SHARED((acc_n, h), jnp.float32),
            pltpu.SemaphoreType.DMA,
        ],
    )
    return call(ms, srcp, dstp, zero)


def _tc_init(degp, x, w0, n):
    def body(degp_ref, x_ref, w0_ref, dinv_ref, ms_ref):
        deg = degp_ref[0] + degp_ref[1]
        dinv = lax.rsqrt(deg[:n, 0:1] + 1.0)
        dinv_ref[...] = dinv
        ms_ref[...] = jnp.dot(x_ref[...] * dinv, w0_ref[...],
                              preferred_element_type=jnp.float32)

    h = x.shape[1]
    return pl.pallas_call(
        body,
        out_shape=(jax.ShapeDtypeStruct((n, 1), jnp.float32),
                   jax.ShapeDtypeStruct((n, h), jnp.float32)),
    )(degp, x, w0)


def _layer_post(p_ref, ms_ref, h_ref, dinv_ref, bc_ref, g_ref, b_ref, n):
    ms = ms_ref[...]
    dinv = dinv_ref[...]
    agg = (p_ref[0, :n] + p_ref[1, :n] + ms) * dinv + bc_ref[...]
    a = jnp.maximum(agg, 0.0)
    mu = jnp.mean(a, axis=0, keepdims=True)
    var = jnp.mean((a - mu) ** 2, axis=0, keepdims=True)
    an = (a - mu) * (g_ref[...] * lax.rsqrt(var + 1e-5)) + b_ref[...]
    return an + h_ref[...]


def _tc_layer(p, ms, hprev, dinv, bci, gi, bi, wnext, n):
    def body(p_ref, ms_ref, h_ref, dinv_ref, bc_ref, g_ref, b_ref, wn_ref,
             hout_ref, msout_ref):
        hn = _layer_post(p_ref, ms_ref, h_ref, dinv_ref, bc_ref, g_ref, b_ref, n)
        hout_ref[...] = hn
        msout_ref[...] = jnp.dot(hn * dinv_ref[...], wn_ref[...],
                                 preferred_element_type=jnp.float32)

    h = ms.shape[1]
    return pl.pallas_call(
        body,
        out_shape=(jax.ShapeDtypeStruct((n, h), jnp.float32),
                   jax.ShapeDtypeStruct((n, h), jnp.float32)),
    )(p, ms, hprev, dinv, bci, gi, bi, wnext)


def _tc_final(p, ms, hprev, dinv, bci, gi, bi, batch_row, w1p, b1p, w2p, b2p, n):
    def body(p_ref, ms_ref, h_ref, dinv_ref, bc_ref, g_ref, b_ref,
             batch_ref, w1_ref, b1_ref, w2_ref, b2_ref, out_ref):
        hn = _layer_post(p_ref, ms_ref, h_ref, dinv_ref, bc_ref, g_ref, b_ref, n)
        seg = lax.broadcasted_iota(jnp.int32, (GSEG, n), 0)
        m = (batch_ref[...] == seg).astype(jnp.float32)
        sums = jnp.dot(m, hn, preferred_element_type=jnp.float32)
        cnt = jnp.sum(m, axis=1, keepdims=True)
        pooled = sums / jnp.maximum(cnt, 1.0)
        z = jnp.maximum(
            jnp.dot(pooled, w1_ref[...], preferred_element_type=jnp.float32)
            + b1_ref[...], 0.0)
        out_ref[...] = jnp.dot(z, w2_ref[...],
                               preferred_element_type=jnp.float32) + b2_ref[...]

    return pl.pallas_call(
        body,
        out_shape=jax.ShapeDtypeStruct((GSEG, 128), jnp.float32),
    )(p, ms, hprev, dinv, bci, gi, bi, batch_row, w1p, b1p, w2p, b2p)


def kernel(x, edge_index, batch, Wc, bc, gamma, beta, W1, b1, W2, b2):
    n, h = x.shape
    e = edge_index.shape[1]
    nlayers = Wc.shape[0]
    hh = W1.shape[1]
    nout = W2.shape[1]

    acc_n = -(-(n + 1) // 128) * 128
    cpw = -(-e // (NW * CB))
    ep = NW * CB * cpw
    pad = ep - e
    srcp = jnp.concatenate([edge_index[0], jnp.zeros((pad,), jnp.int32)])
    dstp = jnp.concatenate([edge_index[1], jnp.full((pad,), n, jnp.int32)])
    zero_acc = jnp.zeros((acc_n, h), jnp.float32)

    w1p = jnp.zeros((h, 128), jnp.float32).at[:, :hh].set(W1)
    b1p = jnp.zeros((1, 128), jnp.float32).at[0, :hh].set(b1)
    w2p = jnp.zeros((128, 128), jnp.float32).at[:hh, :nout].set(W2)
    b2p = jnp.zeros((1, 128), jnp.float32).at[0, :nout].set(b2)
    batch_row = batch.reshape(1, n)

    degp = _sc_degree(dstp, acc_n, cpw, h)
    dinv, ms = _tc_init(degp, x, Wc[0], n)

    hcur = x
    for i in range(nlayers - 1):
        p = _sc_scatter(ms, srcp, dstp, zero_acc, acc_n, cpw, h)
        hcur, ms = _tc_layer(p, ms, hcur, dinv, bc[i].reshape(1, h),
                             gamma[i].reshape(1, h), beta[i].reshape(1, h),
                             Wc[i + 1], n)

    p = _sc_scatter(ms, srcp, dstp, zero_acc, acc_n, cpw, h)
    out_full = _tc_final(p, ms, hcur, dinv, bc[-1].reshape(1, h),
                         gamma[-1].reshape(1, h), beta[-1].reshape(1, h),
                         batch_row, w1p, b1p, w2p, b2p, n)
    return out_full[:, :nout]
